# baseline (speedup 1.0000x reference)
import functools

import jax
import jax.numpy as jnp
from jax import lax
from jax.experimental import pallas as pl
from jax.experimental.pallas import tpu as pltpu
from jax.experimental.pallas import tpu_sc as plsc

N = 10000
E = 320000
D = 128
HD = D // 2
NC = 2
NS = 16
NW = NC * NS
CH = 80
NPAD = 10240
RPT = NPAD // NS
CW = 16
ACW = HD + CW
EPS = 1e-8

EPW_D = E // NW
SCK_D = 25
NSC_D = EPW_D // CH // SCK_D
EPW_M = E // NS
SCK_M = 50
NSC_M = EPW_M // CH // SCK_M

_mesh = plsc.VectorSubcoreMesh(
    core_axis_name="c", subcore_axis_name="s", num_cores=NC)


def _f32(shape):
    return jax.ShapeDtypeStruct(shape, jnp.float32)


def _edge_dot_body(tab, srcg, dstg, scaleg, out,
                   src_v, dst_v, scale_v, out_v, rows_a0, rows_b0,
                   rows_a1, rows_b1, tmp_v, sem_a0, sem_b0, sem_a1, sem_b1):
    c = lax.axis_index("c")
    s = lax.axis_index("s")
    wid = s * NC + c
    lanes = lax.iota(jnp.int32, 16)

    def compute(j, rows_a, rows_b):
        def group(g, _):
            base = 16 * g
            for l in range(16):
                e = base + l
                acc = rows_a[e, pl.ds(0, 16)] * rows_b[e, pl.ds(0, 16)]
                for k in range(1, 8):
                    acc = acc + (rows_a[e, pl.ds(16 * k, 16)]
                                 * rows_b[e, pl.ds(16 * k, 16)])
                tmp_v[l, pl.ds(0, 16)] = acc
            tot = plsc.load_gather(tmp_v, [lanes, jnp.zeros((16,), jnp.int32)])
            for col in range(1, 16):
                tot = tot + plsc.load_gather(
                    tmp_v, [lanes, jnp.full((16,), col, jnp.int32)])
            out_v[j, pl.ds(base, 16)] = tot * scale_v[j, pl.ds(base, 16)]
            return 0

        lax.fori_loop(0, CH // 16, group, 0)

    for t in range(NSC_D):
        pltpu.sync_copy(srcg.at[wid, t], src_v)
        pltpu.sync_copy(dstg.at[wid, t], dst_v)
        pltpu.sync_copy(scaleg.at[wid, t], scale_v)

        pltpu.async_copy(tab.at[src_v.at[0]], rows_a0, sem_a0)
        pltpu.async_copy(tab.at[dst_v.at[0]], rows_b0, sem_b0)

        def pair(i, _):
            j0 = 2 * i
            j1 = j0 + 1
            pltpu.async_copy(tab.at[src_v.at[j1]], rows_a1, sem_a1)
            pltpu.async_copy(tab.at[dst_v.at[j1]], rows_b1, sem_b1)
            pltpu.make_async_copy(tab.at[src_v.at[j0]], rows_a0, sem_a0).wait()
            pltpu.make_async_copy(tab.at[dst_v.at[j0]], rows_b0, sem_b0).wait()
            compute(j0, rows_a0, rows_b0)

            @pl.when(j0 + 2 < SCK_D)
            def _():
                pltpu.async_copy(tab.at[src_v.at[j0 + 2]], rows_a0, sem_a0)
                pltpu.async_copy(tab.at[dst_v.at[j0 + 2]], rows_b0, sem_b0)

            pltpu.make_async_copy(tab.at[src_v.at[j1]], rows_a1, sem_a1).wait()
            pltpu.make_async_copy(tab.at[dst_v.at[j1]], rows_b1, sem_b1).wait()
            compute(j1, rows_a1, rows_b1)
            return 0

        lax.fori_loop(0, SCK_D // 2, pair, 0)
        if SCK_D % 2:
            jl = SCK_D - 1
            pltpu.make_async_copy(tab.at[src_v.at[jl]], rows_a0, sem_a0).wait()
            pltpu.make_async_copy(tab.at[dst_v.at[jl]], rows_b0, sem_b0).wait()
            compute(jl, rows_a0, rows_b0)
        pltpu.sync_copy(out_v, out.at[wid, t])


@jax.jit
def _edge_dot(tab, srcg, dstg, scaleg):
    return pl.kernel(
        _edge_dot_body,
        out_type=_f32((NW, NSC_D, SCK_D, CH)),
        mesh=_mesh,
        scratch_types=[
            pltpu.VMEM((SCK_D, CH), jnp.int32),
            pltpu.VMEM((SCK_D, CH), jnp.int32),
            pltpu.VMEM((SCK_D, CH), jnp.float32),
            pltpu.VMEM((SCK_D, CH), jnp.float32),
            pltpu.VMEM((CH, D), jnp.float32),
            pltpu.VMEM((CH, D), jnp.float32),
            pltpu.VMEM((CH, D), jnp.float32),
            pltpu.VMEM((CH, D), jnp.float32),
            pltpu.VMEM((16, 16), jnp.float32),
            pltpu.SemaphoreType.DMA,
            pltpu.SemaphoreType.DMA,
            pltpu.SemaphoreType.DMA,
            pltpu.SemaphoreType.DMA,
        ],
        compiler_params=pltpu.CompilerParams(needs_layout_passes=False),
    )(tab, srcg, dstg, scaleg)


def _edge_msg_body(tab, srcg, dstg, wg, maskg, msum,
                   src_v, dst_v, w_v, mask_v, rows_v0, rows_v1,
                   msg_v0, msg_v1, zrow_v, acc_rows,
                   semg0, semg1, sems0, sems1):
    c = lax.axis_index("c")
    s = lax.axis_index("s")
    wid = s * NC + c

    def zrow(i, _):
        for k in range(ACW // 16):
            zrow_v[i, pl.ds(16 * k, 16)] = jnp.zeros((16,), jnp.float32)
        return 0

    lax.fori_loop(0, 128, zrow, 0)
    for t in range(RPT // 128):
        pltpu.sync_copy(zrow_v, acc_rows.at[pl.ds(s * RPT + t * 128, 128)])
    plsc.subcore_barrier()

    def compute(j, rows_v, msg_v):
        def group(g, _):
            base = 16 * g
            wv = w_v[j, pl.ds(base, 16)]
            mv = mask_v[j, pl.ds(base, 16)]
            wm = wv * mv
            for l in range(16):
                e = base + l
                w = wm[l]
                for k in range(HD // 16):
                    msg_v[e, pl.ds(16 * k, 16)] = rows_v[e, pl.ds(16 * k, 16)] * w
                msg_v[e, pl.ds(HD, 16)] = jnp.broadcast_to(mv[l], (16,))
            return 0

        lax.fori_loop(0, CH // 16, group, 0)

    for t in range(NSC_M):
        pltpu.sync_copy(srcg.at[wid, t], src_v)
        pltpu.sync_copy(dstg.at[s, t], dst_v)
        pltpu.sync_copy(wg.at[s, t], w_v)
        pltpu.sync_copy(maskg.at[s, t], mask_v)

        pltpu.async_copy(tab.at[src_v.at[0]], rows_v0, semg0)

        def pair(i, _):
            j0 = 2 * i
            j1 = j0 + 1
            pltpu.async_copy(tab.at[src_v.at[j1]], rows_v1, semg1)
            pltpu.make_async_copy(tab.at[src_v.at[j0]], rows_v0, semg0).wait()

            @pl.when(j0 >= 2)
            def _():
                pltpu.make_async_copy(
                    msg_v0, acc_rows.at[dst_v.at[j0]], sems0).wait()

            compute(j0, rows_v0, msg_v0)
            pltpu.async_copy(msg_v0, acc_rows.at[dst_v.at[j0]], sems0, add=True)

            @pl.when(j0 + 2 < SCK_M)
            def _():
                pltpu.async_copy(tab.at[src_v.at[j0 + 2]], rows_v0, semg0)

            pltpu.make_async_copy(tab.at[src_v.at[j1]], rows_v1, semg1).wait()

            @pl.when(j1 >= 2)
            def _():
                pltpu.make_async_copy(
                    msg_v1, acc_rows.at[dst_v.at[j1]], sems1).wait()

            compute(j1, rows_v1, msg_v1)
            pltpu.async_copy(msg_v1, acc_rows.at[dst_v.at[j1]], sems1, add=True)
            return 0

        lax.fori_loop(0, SCK_M // 2, pair, 0)
        pltpu.make_async_copy(msg_v0, acc_rows.at[dst_v.at[0]], sems0).wait()
        pltpu.make_async_copy(msg_v1, acc_rows.at[dst_v.at[0]], sems1).wait()
    plsc.subcore_barrier()
    pltpu.sync_copy(acc_rows.at[pl.ds(s * RPT, RPT)], msum.at[c, pl.ds(s * RPT, RPT)])


@jax.jit
def _edge_msg(tab, srcg, dstg, wg, maskg):
    return pl.kernel(
        _edge_msg_body,
        out_type=_f32((NC, NPAD, ACW)),
        mesh=_mesh,
        scratch_types=[
            pltpu.VMEM((SCK_M, CH), jnp.int32),
            pltpu.VMEM((SCK_M, CH), jnp.int32),
            pltpu.VMEM((SCK_M, CH), jnp.float32),
            pltpu.VMEM((SCK_M, CH), jnp.float32),
            pltpu.VMEM((CH, HD), jnp.float32),
            pltpu.VMEM((CH, HD), jnp.float32),
            pltpu.VMEM((CH, ACW), jnp.float32),
            pltpu.VMEM((CH, ACW), jnp.float32),
            pltpu.VMEM((128, ACW), jnp.float32),
            pltpu.VMEM_SHARED((NPAD, ACW), jnp.float32),
            pltpu.SemaphoreType.DMA,
            pltpu.SemaphoreType.DMA,
            pltpu.SemaphoreType.DMA,
            pltpu.SemaphoreType.DMA,
        ],
        compiler_params=pltpu.CompilerParams(
            needs_layout_passes=False, use_tc_tiling_on_sc=False),
    )(tab, srcg, dstg, wg, maskg)


BR = 1000


def _prep_body(x_ref, wt_ref, mask_ref, un_ref, fo_ref):
    xb = x_ref[...]
    m = mask_ref[...]
    xl = jnp.dot(xb, wt_ref[...], preferred_element_type=jnp.float32)
    fo = m * xl + (1.0 - m) * xb
    fo_ref[0] = fo[:, :HD]
    fo_ref[1] = fo[:, HD:]
    nrm = jnp.maximum(jnp.sqrt(jnp.sum(xb * xb, axis=1, keepdims=True)), EPS)
    un_ref[...] = xb / nrm


@jax.jit
def _prep(x, wt, mask):
    return pl.pallas_call(
        _prep_body,
        grid=(N // BR,),
        in_specs=[
            pl.BlockSpec((BR, D), lambda i: (i, 0)),
            pl.BlockSpec((D, D), lambda i: (0, 0)),
            pl.BlockSpec((BR, 1), lambda i: (i, 0)),
        ],
        out_specs=[
            pl.BlockSpec((BR, D), lambda i: (i, 0)),
            pl.BlockSpec((2, BR, HD), lambda i: (0, i, 0)),
        ],
        out_shape=[_f32((N, D)), _f32((2, N, HD))],
    )(x, wt, mask)


def _post_body(need_un, need_fo, msum_ref, fo_ref, mask_ref, b_ref,
               wt_ref, nmask_ref, *out_refs):
    msum = jnp.concatenate([msum_ref[0, :, :HD], msum_ref[1, :, :HD]], axis=1)
    cnt = msum_ref[0, :, HD:HD + 1]
    mean = jnp.where(cnt > 0.0, msum / jnp.maximum(cnt, 1.0), 0.0)
    m = mask_ref[...]
    fo = jnp.concatenate([fo_ref[0], fo_ref[1]], axis=1)
    out = mean * m + fo
    sig = jax.nn.sigmoid(out + b_ref[...])
    xn = m * sig + (1.0 - m) * out
    i = 0
    if need_fo:
        nm = nmask_ref[...]
        xl = jnp.dot(xn, wt_ref[...], preferred_element_type=jnp.float32)
        fon = nm * xl + (1.0 - nm) * xn
        out_refs[i][0] = fon[:, :HD]
        out_refs[i][1] = fon[:, HD:]
        i += 1
    if need_un:
        nrm = jnp.maximum(jnp.sqrt(jnp.sum(xn * xn, axis=1, keepdims=True)), EPS)
        out_refs[i][...] = xn / nrm


@functools.partial(jax.jit, static_argnames=("need_un", "need_fo"))
def _post(msum_p, fo, mask, b, wt, nmask, need_un, need_fo):
    out_specs = []
    out_shape = []
    if need_fo:
        out_specs.append(pl.BlockSpec((2, BR, HD), lambda i: (0, i, 0)))
        out_shape.append(_f32((2, N, HD)))
    if need_un:
        out_specs.append(pl.BlockSpec((BR, D), lambda i: (i, 0)))
        out_shape.append(_f32((N, D)))
    return pl.pallas_call(
        functools.partial(_post_body, need_un, need_fo),
        grid=(N // BR,),
        in_specs=[
            pl.BlockSpec((NC, BR, ACW), lambda i: (0, i, 0)),
            pl.BlockSpec((2, BR, HD), lambda i: (0, i, 0)),
            pl.BlockSpec((BR, 1), lambda i: (i, 0)),
            pl.BlockSpec((1, D), lambda i: (0, 0)),
            pl.BlockSpec((D, D), lambda i: (0, 0)),
            pl.BlockSpec((BR, 1), lambda i: (i, 0)),
        ],
        out_specs=out_specs,
        out_shape=out_shape,
    )(msum_p, fo, mask, b, wt, nmask)


def kernel(x, edge_attr, W1_ii, b1_ii, W2_ii, b2_ii, W1_uiu, b1_uiu,
           edge_index, edge_mask_ii, edge_mask_uiu, node_mask_item):
    src_d = edge_index[0].reshape(NW, NSC_D, SCK_D, CH)
    dst_d = edge_index[1].reshape(NW, NSC_D, SCK_D, CH)
    ea_d = edge_attr.reshape(NW, NSC_D, SCK_D, CH)
    src_m = edge_index[0].reshape(NS, 1, NSC_M, SCK_M, CH)
    coff = (jnp.arange(NC, dtype=jnp.int32) * N).reshape(1, NC, 1, 1, 1)
    src_m = (src_m + coff).reshape(NW, NSC_M, SCK_M, CH)
    dst_m = edge_index[1].reshape(NS, NSC_M, SCK_M, CH)
    ea_m = edge_attr.reshape(NS, NSC_M, SCK_M, CH)
    mii_m = edge_mask_ii.astype(jnp.float32).reshape(NS, NSC_M, SCK_M, CH)
    muiu_m = edge_mask_uiu.astype(jnp.float32).reshape(NS, NSC_M, SCK_M, CH)
    muiu_d = edge_mask_uiu.astype(jnp.float32).reshape(NW, NSC_D, SCK_D, CH)
    nmask = node_mask_item.astype(jnp.float32)[:, None]
    allmask = jnp.ones((N, 1), jnp.float32)

    b1i = b1_ii.reshape(1, D)
    b2i = b2_ii.reshape(1, D)
    b1u = b1_uiu.reshape(1, D)
    w1t = W1_ii.T
    w2t = W2_ii.T
    wut = W1_uiu.T

    un0, fo1 = _prep(x, w1t, nmask)
    w1 = _edge_dot(un0, src_d, dst_d, ea_d)
    ms1 = _edge_msg(fo1.reshape(2 * N, HD), src_m, dst_m,
                    w1.reshape(NS, NSC_M, SCK_M, CH), mii_m)
    fo2, un1 = _post(ms1, fo1, nmask, b1i, w2t, nmask,
                     need_un=True, need_fo=True)
    w2 = _edge_dot(un1, src_d, dst_d, ea_d)
    ms2 = _edge_msg(fo2.reshape(2 * N, HD), src_m, dst_m,
                    w2.reshape(NS, NSC_M, SCK_M, CH), mii_m)
    (fo3,) = _post(ms2, fo2, nmask, b2i, wut, allmask,
                   need_un=False, need_fo=True)
    ms3 = _edge_msg(fo3.reshape(2 * N, HD), src_m, dst_m, ea_m, muiu_m)
    (fo4,) = _post(ms3, fo3, allmask, b1u, wut, allmask,
                   need_un=False, need_fo=True)
    ms4 = _edge_msg(fo4.reshape(2 * N, HD), src_m, dst_m, ea_m, muiu_m)
    (un4,) = _post(ms4, fo4, allmask, b1u, wut, allmask,
                   need_un=True, need_fo=False)
    y = _edge_dot(un4, src_d, dst_d, muiu_d)
    return y.reshape(E)


# speedup vs baseline: 1.2941x; 1.2941x over previous
import functools

import jax
import jax.numpy as jnp
from jax import lax
from jax.experimental import pallas as pl
from jax.experimental.pallas import tpu as pltpu
from jax.experimental.pallas import tpu_sc as plsc

N = 10000
E = 320000
D = 128
HD = D // 2
NC = 2
NS = 16
NW = NC * NS
CH = 80
NPAD = 10240
RPT = NPAD // NS
CW = 16
ACW = HD + CW
EPS = 1e-8

EPW_D = E // NW
SCK_D = 25
NSC_D = EPW_D // CH // SCK_D
EPW_M = E // NS
SCK_M = 50
NSC_M = EPW_M // CH // SCK_M

_mesh = plsc.VectorSubcoreMesh(
    core_axis_name="c", subcore_axis_name="s", num_cores=NC)


def _f32(shape):
    return jax.ShapeDtypeStruct(shape, jnp.float32)


def _edge_dot_body(tab, srcg, dstg, scaleg, out,
                   src_v, dst_v, scale_v, out_v, rows_a0, rows_b0,
                   rows_a1, rows_b1, tmp_v, sem_a0, sem_b0, sem_a1, sem_b1):
    c = lax.axis_index("c")
    s = lax.axis_index("s")
    wid = s * NC + c
    lanes = lax.iota(jnp.int32, 16)

    def compute(j, rows_a, rows_b):
        def group(g, _):
            base = 16 * g
            for l in range(16):
                e = base + l
                acc = rows_a[e, pl.ds(0, 16)] * rows_b[e, pl.ds(0, 16)]
                for k in range(1, 8):
                    acc = acc + (rows_a[e, pl.ds(16 * k, 16)]
                                 * rows_b[e, pl.ds(16 * k, 16)])
                tmp_v[l, pl.ds(0, 16)] = acc
            tot = plsc.load_gather(tmp_v, [lanes, jnp.zeros((16,), jnp.int32)])
            for col in range(1, 16):
                tot = tot + plsc.load_gather(
                    tmp_v, [lanes, jnp.full((16,), col, jnp.int32)])
            out_v[j, pl.ds(base, 16)] = tot * scale_v[j, pl.ds(base, 16)]
            return 0

        lax.fori_loop(0, CH // 16, group, 0)

    for t in range(NSC_D):
        pltpu.sync_copy(srcg.at[wid, t], src_v)
        pltpu.sync_copy(dstg.at[wid, t], dst_v)
        pltpu.sync_copy(scaleg.at[wid, t], scale_v)

        pltpu.async_copy(tab.at[src_v.at[0]], rows_a0, sem_a0)
        pltpu.async_copy(tab.at[dst_v.at[0]], rows_b0, sem_b0)

        def pair(i, _):
            j0 = 2 * i
            j1 = j0 + 1
            pltpu.async_copy(tab.at[src_v.at[j1]], rows_a1, sem_a1)
            pltpu.async_copy(tab.at[dst_v.at[j1]], rows_b1, sem_b1)
            pltpu.make_async_copy(tab.at[src_v.at[j0]], rows_a0, sem_a0).wait()
            pltpu.make_async_copy(tab.at[dst_v.at[j0]], rows_b0, sem_b0).wait()
            compute(j0, rows_a0, rows_b0)

            @pl.when(j0 + 2 < SCK_D)
            def _():
                pltpu.async_copy(tab.at[src_v.at[j0 + 2]], rows_a0, sem_a0)
                pltpu.async_copy(tab.at[dst_v.at[j0 + 2]], rows_b0, sem_b0)

            pltpu.make_async_copy(tab.at[src_v.at[j1]], rows_a1, sem_a1).wait()
            pltpu.make_async_copy(tab.at[dst_v.at[j1]], rows_b1, sem_b1).wait()
            compute(j1, rows_a1, rows_b1)
            return 0

        lax.fori_loop(0, SCK_D // 2, pair, 0)
        if SCK_D % 2:
            jl = SCK_D - 1
            pltpu.make_async_copy(tab.at[src_v.at[jl]], rows_a0, sem_a0).wait()
            pltpu.make_async_copy(tab.at[dst_v.at[jl]], rows_b0, sem_b0).wait()
            compute(jl, rows_a0, rows_b0)
        pltpu.sync_copy(out_v, out.at[wid, t])


@jax.jit
def _edge_dot(tab, srcg, dstg, scaleg):
    return pl.kernel(
        _edge_dot_body,
        out_type=_f32((NW, NSC_D, SCK_D, CH)),
        mesh=_mesh,
        scratch_types=[
            pltpu.VMEM((SCK_D, CH), jnp.int32),
            pltpu.VMEM((SCK_D, CH), jnp.int32),
            pltpu.VMEM((SCK_D, CH), jnp.float32),
            pltpu.VMEM((SCK_D, CH), jnp.float32),
            pltpu.VMEM((CH, D), jnp.float32),
            pltpu.VMEM((CH, D), jnp.float32),
            pltpu.VMEM((CH, D), jnp.float32),
            pltpu.VMEM((CH, D), jnp.float32),
            pltpu.VMEM((16, 16), jnp.float32),
            pltpu.SemaphoreType.DMA,
            pltpu.SemaphoreType.DMA,
            pltpu.SemaphoreType.DMA,
            pltpu.SemaphoreType.DMA,
        ],
        compiler_params=pltpu.CompilerParams(needs_layout_passes=False),
    )(tab, srcg, dstg, scaleg)


def _edge_msg_body(with_cnt, tab, srcg, dstg, wg, maskg, *rest):
    if with_cnt:
        (msum, cntp, src_v, dst_v, w_v, mask_v, rows_v0, rows_v1, cnt_v,
         zrow_v, zcnt_v, acc_rows, acc_cnt, sem0, sem1) = rest
    else:
        (msum, src_v, dst_v, w_v, mask_v, rows_v0, rows_v1,
         zrow_v, acc_rows, sem0, sem1) = rest
    c = lax.axis_index("c")
    s = lax.axis_index("s")
    wid = s * NC + c

    def zrow(i, _):
        for k in range(HD // 16):
            zrow_v[i, pl.ds(16 * k, 16)] = jnp.zeros((16,), jnp.float32)
        if with_cnt:
            zcnt_v[i, pl.ds(0, 16)] = jnp.zeros((16,), jnp.float32)
        return 0

    lax.fori_loop(0, 128, zrow, 0)
    for t in range(RPT // 128):
        pltpu.sync_copy(zrow_v, acc_rows.at[pl.ds(s * RPT + t * 128, 128)])
        if with_cnt:
            pltpu.sync_copy(zcnt_v, acc_cnt.at[pl.ds(s * RPT + t * 128, 128)])
    plsc.subcore_barrier()

    def compute_scatter(j, rows_v):
        def group(g, _):
            base = 16 * g
            wv = w_v[j, pl.ds(base, 16)]
            mv = mask_v[j, pl.ds(base, 16)]
            wm = wv * mv
            for l in range(16):
                e = base + l
                w = wm[l]
                for k in range(HD // 16):
                    rows_v[e, pl.ds(16 * k, 16)] = rows_v[e, pl.ds(16 * k, 16)] * w
                if with_cnt:
                    cnt_v[e, pl.ds(0, 16)] = jnp.broadcast_to(mv[l], (16,))
            return 0

        lax.fori_loop(0, CH // 16, group, 0)
        pltpu.sync_copy(rows_v, acc_rows.at[dst_v.at[j]], add=True)
        if with_cnt:
            pltpu.sync_copy(cnt_v, acc_cnt.at[dst_v.at[j]], add=True)

    for t in range(NSC_M):
        pltpu.sync_copy(srcg.at[wid, t], src_v)
        pltpu.sync_copy(dstg.at[s, t], dst_v)
        pltpu.sync_copy(wg.at[s, t], w_v)
        pltpu.sync_copy(maskg.at[s, t], mask_v)

        pltpu.async_copy(tab.at[src_v.at[0]], rows_v0, sem0)

        def pair(i, _):
            j0 = 2 * i
            j1 = j0 + 1
            pltpu.async_copy(tab.at[src_v.at[j1]], rows_v1, sem1)
            pltpu.make_async_copy(tab.at[src_v.at[j0]], rows_v0, sem0).wait()
            compute_scatter(j0, rows_v0)

            @pl.when(j0 + 2 < SCK_M)
            def _():
                pltpu.async_copy(tab.at[src_v.at[j0 + 2]], rows_v0, sem0)

            pltpu.make_async_copy(tab.at[src_v.at[j1]], rows_v1, sem1).wait()
            compute_scatter(j1, rows_v1)
            return 0

        lax.fori_loop(0, SCK_M // 2, pair, 0)
    plsc.subcore_barrier()
    pltpu.sync_copy(acc_rows.at[pl.ds(s * RPT, RPT)], msum.at[c, pl.ds(s * RPT, RPT)])
    if with_cnt:
        pltpu.sync_copy(acc_cnt.at[pl.ds(s * RPT, RPT)], cntp.at[c, pl.ds(s * RPT, RPT)])


@functools.partial(jax.jit, static_argnames=("with_cnt",))
def _edge_msg(tab, srcg, dstg, wg, maskg, with_cnt):
    if with_cnt:
        out_type = (_f32((NC, NPAD, HD)), _f32((NC, NPAD, CW)))
        scratch = [
            pltpu.VMEM((SCK_M, CH), jnp.int32),
            pltpu.VMEM((SCK_M, CH), jnp.int32),
            pltpu.VMEM((SCK_M, CH), jnp.float32),
            pltpu.VMEM((SCK_M, CH), jnp.float32),
            pltpu.VMEM((CH, HD), jnp.float32),
            pltpu.VMEM((CH, HD), jnp.float32),
            pltpu.VMEM((CH, CW), jnp.float32),
            pltpu.VMEM((128, HD), jnp.float32),
            pltpu.VMEM((128, CW), jnp.float32),
            pltpu.VMEM_SHARED((NPAD, HD), jnp.float32),
            pltpu.VMEM_SHARED((NPAD, CW), jnp.float32),
            pltpu.SemaphoreType.DMA,
            pltpu.SemaphoreType.DMA,
        ]
    else:
        out_type = _f32((NC, NPAD, HD))
        scratch = [
            pltpu.VMEM((SCK_M, CH), jnp.int32),
            pltpu.VMEM((SCK_M, CH), jnp.int32),
            pltpu.VMEM((SCK_M, CH), jnp.float32),
            pltpu.VMEM((SCK_M, CH), jnp.float32),
            pltpu.VMEM((CH, HD), jnp.float32),
            pltpu.VMEM((CH, HD), jnp.float32),
            pltpu.VMEM((128, HD), jnp.float32),
            pltpu.VMEM_SHARED((NPAD, HD), jnp.float32),
            pltpu.SemaphoreType.DMA,
            pltpu.SemaphoreType.DMA,
        ]
    return pl.kernel(
        functools.partial(_edge_msg_body, with_cnt),
        out_type=out_type,
        mesh=_mesh,
        scratch_types=scratch,
        compiler_params=pltpu.CompilerParams(
            needs_layout_passes=False, use_tc_tiling_on_sc=False),
    )(tab, srcg, dstg, wg, maskg)


BR = 1000


def _prep_body(x_ref, wt_ref, mask_ref, un_ref, fo_ref):
    xb = x_ref[...]
    m = mask_ref[...]
    xl = jnp.dot(xb, wt_ref[...], preferred_element_type=jnp.float32)
    fo = m * xl + (1.0 - m) * xb
    fo_ref[0] = fo[:, :HD]
    fo_ref[1] = fo[:, HD:]
    nrm = jnp.maximum(jnp.sqrt(jnp.sum(xb * xb, axis=1, keepdims=True)), EPS)
    un_ref[...] = xb / nrm


@jax.jit
def _prep(x, wt, mask):
    return pl.pallas_call(
        _prep_body,
        grid=(N // BR,),
        in_specs=[
            pl.BlockSpec((BR, D), lambda i: (i, 0)),
            pl.BlockSpec((D, D), lambda i: (0, 0)),
            pl.BlockSpec((BR, 1), lambda i: (i, 0)),
        ],
        out_specs=[
            pl.BlockSpec((BR, D), lambda i: (i, 0)),
            pl.BlockSpec((2, BR, HD), lambda i: (0, i, 0)),
        ],
        out_shape=[_f32((N, D)), _f32((2, N, HD))],
    )(x, wt, mask)


def _post_body(need_un, need_fo, msum_ref, cnt_ref, fo_ref, mask_ref, b_ref,
               wt_ref, nmask_ref, *out_refs):
    msum = jnp.concatenate([msum_ref[0], msum_ref[1]], axis=1)
    cnt = cnt_ref[0, :, 0:1]
    mean = jnp.where(cnt > 0.0, msum / jnp.maximum(cnt, 1.0), 0.0)
    m = mask_ref[...]
    fo = jnp.concatenate([fo_ref[0], fo_ref[1]], axis=1)
    out = mean * m + fo
    sig = jax.nn.sigmoid(out + b_ref[...])
    xn = m * sig + (1.0 - m) * out
    i = 0
    if need_fo:
        nm = nmask_ref[...]
        xl = jnp.dot(xn, wt_ref[...], preferred_element_type=jnp.float32)
        fon = nm * xl + (1.0 - nm) * xn
        out_refs[i][0] = fon[:, :HD]
        out_refs[i][1] = fon[:, HD:]
        i += 1
    if need_un:
        nrm = jnp.maximum(jnp.sqrt(jnp.sum(xn * xn, axis=1, keepdims=True)), EPS)
        out_refs[i][...] = xn / nrm


@functools.partial(jax.jit, static_argnames=("need_un", "need_fo"))
def _post(msum_p, cnt_p, fo, mask, b, wt, nmask, need_un, need_fo):
    out_specs = []
    out_shape = []
    if need_fo:
        out_specs.append(pl.BlockSpec((2, BR, HD), lambda i: (0, i, 0)))
        out_shape.append(_f32((2, N, HD)))
    if need_un:
        out_specs.append(pl.BlockSpec((BR, D), lambda i: (i, 0)))
        out_shape.append(_f32((N, D)))
    return pl.pallas_call(
        functools.partial(_post_body, need_un, need_fo),
        grid=(N // BR,),
        in_specs=[
            pl.BlockSpec((NC, BR, HD), lambda i: (0, i, 0)),
            pl.BlockSpec((NC, BR, CW), lambda i: (0, i, 0)),
            pl.BlockSpec((2, BR, HD), lambda i: (0, i, 0)),
            pl.BlockSpec((BR, 1), lambda i: (i, 0)),
            pl.BlockSpec((1, D), lambda i: (0, 0)),
            pl.BlockSpec((D, D), lambda i: (0, 0)),
            pl.BlockSpec((BR, 1), lambda i: (i, 0)),
        ],
        out_specs=out_specs,
        out_shape=out_shape,
    )(msum_p, cnt_p, fo, mask, b, wt, nmask)


def kernel(x, edge_attr, W1_ii, b1_ii, W2_ii, b2_ii, W1_uiu, b1_uiu,
           edge_index, edge_mask_ii, edge_mask_uiu, node_mask_item):
    src_d = edge_index[0].reshape(NW, NSC_D, SCK_D, CH)
    dst_d = edge_index[1].reshape(NW, NSC_D, SCK_D, CH)
    ea_d = edge_attr.reshape(NW, NSC_D, SCK_D, CH)
    src_m = edge_index[0].reshape(NS, 1, NSC_M, SCK_M, CH)
    coff = (jnp.arange(NC, dtype=jnp.int32) * N).reshape(1, NC, 1, 1, 1)
    src_m = (src_m + coff).reshape(NW, NSC_M, SCK_M, CH)
    dst_m = edge_index[1].reshape(NS, NSC_M, SCK_M, CH)
    ea_m = edge_attr.reshape(NS, NSC_M, SCK_M, CH)
    mii_m = edge_mask_ii.astype(jnp.float32).reshape(NS, NSC_M, SCK_M, CH)
    muiu_m = edge_mask_uiu.astype(jnp.float32).reshape(NS, NSC_M, SCK_M, CH)
    muiu_d = edge_mask_uiu.astype(jnp.float32).reshape(NW, NSC_D, SCK_D, CH)
    nmask = node_mask_item.astype(jnp.float32)[:, None]
    allmask = jnp.ones((N, 1), jnp.float32)

    b1i = b1_ii.reshape(1, D)
    b2i = b2_ii.reshape(1, D)
    b1u = b1_uiu.reshape(1, D)
    w1t = W1_ii.T
    w2t = W2_ii.T
    wut = W1_uiu.T

    un0, fo1 = _prep(x, w1t, nmask)
    w1 = _edge_dot(un0, src_d, dst_d, ea_d)
    ms1, ct_ii = _edge_msg(fo1.reshape(2 * N, HD), src_m, dst_m,
                           w1.reshape(NS, NSC_M, SCK_M, CH), mii_m,
                           with_cnt=True)
    fo2, un1 = _post(ms1, ct_ii, fo1, nmask, b1i, w2t, nmask,
                     need_un=True, need_fo=True)
    w2 = _edge_dot(un1, src_d, dst_d, ea_d)
    ms2 = _edge_msg(fo2.reshape(2 * N, HD), src_m, dst_m,
                    w2.reshape(NS, NSC_M, SCK_M, CH), mii_m, with_cnt=False)
    (fo3,) = _post(ms2, ct_ii, fo2, nmask, b2i, wut, allmask,
                   need_un=False, need_fo=True)
    ms3, ct_uiu = _edge_msg(fo3.reshape(2 * N, HD), src_m, dst_m, ea_m,
                            muiu_m, with_cnt=True)
    (fo4,) = _post(ms3, ct_uiu, fo3, allmask, b1u, wut, allmask,
                   need_un=False, need_fo=True)
    ms4 = _edge_msg(fo4.reshape(2 * N, HD), src_m, dst_m, ea_m, muiu_m,
                    with_cnt=False)
    (un4,) = _post(ms4, ct_uiu, fo4, allmask, b1u, wut, allmask,
                   need_un=True, need_fo=False)
    y = _edge_dot(un4, src_d, dst_d, muiu_d)
    return y.reshape(E)


# speedup vs baseline: 1.3159x; 1.0168x over previous
import functools

import jax
import jax.numpy as jnp
from jax import lax
from jax.experimental import pallas as pl
from jax.experimental.pallas import tpu as pltpu
from jax.experimental.pallas import tpu_sc as plsc

N = 10000
E = 320000
D = 128
HD = D // 2
NC = 2
NS = 16
NW = NC * NS
CH = 80
NPAD = 10240
RPT = NPAD // NS
CW = 16
ACW = HD + CW
EPS = 1e-8

EPW_D = E // NW
SCK_D = 25
NSC_D = EPW_D // CH // SCK_D
EPW_M = E // NS
SCK_M = 50
NSC_M = EPW_M // CH // SCK_M

_mesh = plsc.VectorSubcoreMesh(
    core_axis_name="c", subcore_axis_name="s", num_cores=NC)


def _f32(shape):
    return jax.ShapeDtypeStruct(shape, jnp.float32)


def _edge_dot_body(tab, srcg, dstg, scaleg, out,
                   src_v, dst_v, scale_v, out_v, rows_a0, rows_b0,
                   rows_a1, rows_b1, tmp_v, sem_a0, sem_b0, sem_a1, sem_b1):
    c = lax.axis_index("c")
    s = lax.axis_index("s")
    wid = s * NC + c
    lanes = lax.iota(jnp.int32, 16)

    def compute(j, rows_a, rows_b):
        def group(g, _):
            base = 16 * g
            for l in range(16):
                e = base + l
                acc = rows_a[e, pl.ds(0, 16)] * rows_b[e, pl.ds(0, 16)]
                for k in range(1, 8):
                    acc = acc + (rows_a[e, pl.ds(16 * k, 16)]
                                 * rows_b[e, pl.ds(16 * k, 16)])
                tmp_v[l, pl.ds(0, 16)] = acc
            tot = plsc.load_gather(tmp_v, [lanes, jnp.zeros((16,), jnp.int32)])
            for col in range(1, 16):
                tot = tot + plsc.load_gather(
                    tmp_v, [lanes, jnp.full((16,), col, jnp.int32)])
            out_v[j, pl.ds(base, 16)] = tot * scale_v[j, pl.ds(base, 16)]
            return 0

        lax.fori_loop(0, CH // 16, group, 0)

    for t in range(NSC_D):
        pltpu.sync_copy(srcg.at[wid, t], src_v)
        pltpu.sync_copy(dstg.at[wid, t], dst_v)
        pltpu.sync_copy(scaleg.at[wid, t], scale_v)

        pltpu.async_copy(tab.at[src_v.at[0]], rows_a0, sem_a0)
        pltpu.async_copy(tab.at[dst_v.at[0]], rows_b0, sem_b0)

        def pair(i, _):
            j0 = 2 * i
            j1 = j0 + 1
            pltpu.async_copy(tab.at[src_v.at[j1]], rows_a1, sem_a1)
            pltpu.async_copy(tab.at[dst_v.at[j1]], rows_b1, sem_b1)
            pltpu.make_async_copy(tab.at[src_v.at[j0]], rows_a0, sem_a0).wait()
            pltpu.make_async_copy(tab.at[dst_v.at[j0]], rows_b0, sem_b0).wait()
            compute(j0, rows_a0, rows_b0)

            @pl.when(j0 + 2 < SCK_D)
            def _():
                pltpu.async_copy(tab.at[src_v.at[j0 + 2]], rows_a0, sem_a0)
                pltpu.async_copy(tab.at[dst_v.at[j0 + 2]], rows_b0, sem_b0)

            pltpu.make_async_copy(tab.at[src_v.at[j1]], rows_a1, sem_a1).wait()
            pltpu.make_async_copy(tab.at[dst_v.at[j1]], rows_b1, sem_b1).wait()
            compute(j1, rows_a1, rows_b1)
            return 0

        lax.fori_loop(0, SCK_D // 2, pair, 0)
        if SCK_D % 2:
            jl = SCK_D - 1
            pltpu.make_async_copy(tab.at[src_v.at[jl]], rows_a0, sem_a0).wait()
            pltpu.make_async_copy(tab.at[dst_v.at[jl]], rows_b0, sem_b0).wait()
            compute(jl, rows_a0, rows_b0)
        pltpu.sync_copy(out_v, out.at[wid, t])


@jax.jit
def _edge_dot(tab, srcg, dstg, scaleg):
    return pl.kernel(
        _edge_dot_body,
        out_type=_f32((NW, NSC_D, SCK_D, CH)),
        mesh=_mesh,
        scratch_types=[
            pltpu.VMEM((SCK_D, CH), jnp.int32),
            pltpu.VMEM((SCK_D, CH), jnp.int32),
            pltpu.VMEM((SCK_D, CH), jnp.float32),
            pltpu.VMEM((SCK_D, CH), jnp.float32),
            pltpu.VMEM((CH, D), jnp.float32),
            pltpu.VMEM((CH, D), jnp.float32),
            pltpu.VMEM((CH, D), jnp.float32),
            pltpu.VMEM((CH, D), jnp.float32),
            pltpu.VMEM((16, 16), jnp.float32),
            pltpu.SemaphoreType.DMA,
            pltpu.SemaphoreType.DMA,
            pltpu.SemaphoreType.DMA,
            pltpu.SemaphoreType.DMA,
        ],
        compiler_params=pltpu.CompilerParams(needs_layout_passes=False),
    )(tab, srcg, dstg, scaleg)


def _edge_msg_body(with_cnt, tab, srcg, dstg, wg, maskg, *rest):
    if with_cnt:
        (msum, cntp, src_v, dst_v, w_v, mask_v, rows_v0, rows_v1, cnt_v,
         zrow_v, zcnt_v, acc_rows, acc_cnt, sem0, sem1, sems0, sems1) = rest
    else:
        (msum, src_v, dst_v, w_v, mask_v, rows_v0, rows_v1,
         zrow_v, acc_rows, sem0, sem1, sems0, sems1) = rest
    c = lax.axis_index("c")
    s = lax.axis_index("s")
    wid = s * NC + c

    def zrow(i, _):
        for k in range(HD // 16):
            zrow_v[i, pl.ds(16 * k, 16)] = jnp.zeros((16,), jnp.float32)
        if with_cnt:
            zcnt_v[i, pl.ds(0, 16)] = jnp.zeros((16,), jnp.float32)
        return 0

    lax.fori_loop(0, 128, zrow, 0)
    for t in range(RPT // 128):
        pltpu.sync_copy(zrow_v, acc_rows.at[pl.ds(s * RPT + t * 128, 128)])
        if with_cnt:
            pltpu.sync_copy(zcnt_v, acc_cnt.at[pl.ds(s * RPT + t * 128, 128)])
    plsc.subcore_barrier()

    def compute_scatter(j, rows_v, sems):
        def group(g, _):
            base = 16 * g
            wv = w_v[j, pl.ds(base, 16)]
            mv = mask_v[j, pl.ds(base, 16)]
            wm = wv * mv
            for l in range(16):
                e = base + l
                w = wm[l]
                for k in range(HD // 16):
                    rows_v[e, pl.ds(16 * k, 16)] = rows_v[e, pl.ds(16 * k, 16)] * w
                if with_cnt:
                    cnt_v[e, pl.ds(0, 16)] = jnp.broadcast_to(mv[l], (16,))
            return 0

        lax.fori_loop(0, CH // 16, group, 0)
        pltpu.async_copy(rows_v, acc_rows.at[dst_v.at[j]], sems, add=True)
        if with_cnt:
            pltpu.sync_copy(cnt_v, acc_cnt.at[dst_v.at[j]], add=True)

    for t in range(NSC_M):
        pltpu.sync_copy(srcg.at[wid, t], src_v)
        pltpu.sync_copy(dstg.at[s, t], dst_v)
        pltpu.sync_copy(wg.at[s, t], w_v)
        pltpu.sync_copy(maskg.at[s, t], mask_v)

        pltpu.async_copy(tab.at[src_v.at[0]], rows_v0, sem0)
        pltpu.async_copy(tab.at[src_v.at[1]], rows_v1, sem1)

        def pair(i, _):
            j0 = 2 * i
            j1 = j0 + 1
            pltpu.make_async_copy(tab.at[src_v.at[j0]], rows_v0, sem0).wait()
            compute_scatter(j0, rows_v0, sems0)
            pltpu.make_async_copy(tab.at[src_v.at[j1]], rows_v1, sem1).wait()
            pltpu.make_async_copy(rows_v0, acc_rows.at[dst_v.at[j0]], sems0).wait()

            @pl.when(j0 + 2 < SCK_M)
            def _():
                pltpu.async_copy(tab.at[src_v.at[j0 + 2]], rows_v0, sem0)

            compute_scatter(j1, rows_v1, sems1)
            pltpu.make_async_copy(rows_v1, acc_rows.at[dst_v.at[j1]], sems1).wait()

            @pl.when(j1 + 2 < SCK_M)
            def _():
                pltpu.async_copy(tab.at[src_v.at[j1 + 2]], rows_v1, sem1)

            return 0

        lax.fori_loop(0, SCK_M // 2, pair, 0)
    plsc.subcore_barrier()
    pltpu.sync_copy(acc_rows.at[pl.ds(s * RPT, RPT)], msum.at[c, pl.ds(s * RPT, RPT)])
    if with_cnt:
        pltpu.sync_copy(acc_cnt.at[pl.ds(s * RPT, RPT)], cntp.at[c, pl.ds(s * RPT, RPT)])


@functools.partial(jax.jit, static_argnames=("with_cnt",))
def _edge_msg(tab, srcg, dstg, wg, maskg, with_cnt):
    if with_cnt:
        out_type = (_f32((NC, NPAD, HD)), _f32((NC, NPAD, CW)))
        scratch = [
            pltpu.VMEM((SCK_M, CH), jnp.int32),
            pltpu.VMEM((SCK_M, CH), jnp.int32),
            pltpu.VMEM((SCK_M, CH), jnp.float32),
            pltpu.VMEM((SCK_M, CH), jnp.float32),
            pltpu.VMEM((CH, HD), jnp.float32),
            pltpu.VMEM((CH, HD), jnp.float32),
            pltpu.VMEM((CH, CW), jnp.float32),
            pltpu.VMEM((128, HD), jnp.float32),
            pltpu.VMEM((128, CW), jnp.float32),
            pltpu.VMEM_SHARED((NPAD, HD), jnp.float32),
            pltpu.VMEM_SHARED((NPAD, CW), jnp.float32),
            pltpu.SemaphoreType.DMA,
            pltpu.SemaphoreType.DMA,
            pltpu.SemaphoreType.DMA,
            pltpu.SemaphoreType.DMA,
        ]
    else:
        out_type = _f32((NC, NPAD, HD))
        scratch = [
            pltpu.VMEM((SCK_M, CH), jnp.int32),
            pltpu.VMEM((SCK_M, CH), jnp.int32),
            pltpu.VMEM((SCK_M, CH), jnp.float32),
            pltpu.VMEM((SCK_M, CH), jnp.float32),
            pltpu.VMEM((CH, HD), jnp.float32),
            pltpu.VMEM((CH, HD), jnp.float32),
            pltpu.VMEM((128, HD), jnp.float32),
            pltpu.VMEM_SHARED((NPAD, HD), jnp.float32),
            pltpu.SemaphoreType.DMA,
            pltpu.SemaphoreType.DMA,
            pltpu.SemaphoreType.DMA,
            pltpu.SemaphoreType.DMA,
        ]
    return pl.kernel(
        functools.partial(_edge_msg_body, with_cnt),
        out_type=out_type,
        mesh=_mesh,
        scratch_types=scratch,
        compiler_params=pltpu.CompilerParams(
            needs_layout_passes=False, use_tc_tiling_on_sc=False),
    )(tab, srcg, dstg, wg, maskg)


BR = 1000


def _prep_body(x_ref, wt_ref, mask_ref, un_ref, fo_ref):
    xb = x_ref[...]
    m = mask_ref[...]
    xl = jnp.dot(xb, wt_ref[...], preferred_element_type=jnp.float32)
    fo = m * xl + (1.0 - m) * xb
    fo_ref[0] = fo[:, :HD]
    fo_ref[1] = fo[:, HD:]
    nrm = jnp.maximum(jnp.sqrt(jnp.sum(xb * xb, axis=1, keepdims=True)), EPS)
    un_ref[...] = xb / nrm


@jax.jit
def _prep(x, wt, mask):
    return pl.pallas_call(
        _prep_body,
        grid=(N // BR,),
        in_specs=[
            pl.BlockSpec((BR, D), lambda i: (i, 0)),
            pl.BlockSpec((D, D), lambda i: (0, 0)),
            pl.BlockSpec((BR, 1), lambda i: (i, 0)),
        ],
        out_specs=[
            pl.BlockSpec((BR, D), lambda i: (i, 0)),
            pl.BlockSpec((2, BR, HD), lambda i: (0, i, 0)),
        ],
        out_shape=[_f32((N, D)), _f32((2, N, HD))],
    )(x, wt, mask)


def _post_body(need_un, need_fo, msum_ref, cnt_ref, fo_ref, mask_ref, b_ref,
               wt_ref, nmask_ref, *out_refs):
    msum = jnp.concatenate([msum_ref[0], msum_ref[1]], axis=1)
    cnt = cnt_ref[0, :, 0:1]
    mean = jnp.where(cnt > 0.0, msum / jnp.maximum(cnt, 1.0), 0.0)
    m = mask_ref[...]
    fo = jnp.concatenate([fo_ref[0], fo_ref[1]], axis=1)
    out = mean * m + fo
    sig = jax.nn.sigmoid(out + b_ref[...])
    xn = m * sig + (1.0 - m) * out
    i = 0
    if need_fo:
        nm = nmask_ref[...]
        xl = jnp.dot(xn, wt_ref[...], preferred_element_type=jnp.float32)
        fon = nm * xl + (1.0 - nm) * xn
        out_refs[i][0] = fon[:, :HD]
        out_refs[i][1] = fon[:, HD:]
        i += 1
    if need_un:
        nrm = jnp.maximum(jnp.sqrt(jnp.sum(xn * xn, axis=1, keepdims=True)), EPS)
        out_refs[i][...] = xn / nrm


@functools.partial(jax.jit, static_argnames=("need_un", "need_fo"))
def _post(msum_p, cnt_p, fo, mask, b, wt, nmask, need_un, need_fo):
    out_specs = []
    out_shape = []
    if need_fo:
        out_specs.append(pl.BlockSpec((2, BR, HD), lambda i: (0, i, 0)))
        out_shape.append(_f32((2, N, HD)))
    if need_un:
        out_specs.append(pl.BlockSpec((BR, D), lambda i: (i, 0)))
        out_shape.append(_f32((N, D)))
    return pl.pallas_call(
        functools.partial(_post_body, need_un, need_fo),
        grid=(N // BR,),
        in_specs=[
            pl.BlockSpec((NC, BR, HD), lambda i: (0, i, 0)),
            pl.BlockSpec((NC, BR, CW), lambda i: (0, i, 0)),
            pl.BlockSpec((2, BR, HD), lambda i: (0, i, 0)),
            pl.BlockSpec((BR, 1), lambda i: (i, 0)),
            pl.BlockSpec((1, D), lambda i: (0, 0)),
            pl.BlockSpec((D, D), lambda i: (0, 0)),
            pl.BlockSpec((BR, 1), lambda i: (i, 0)),
        ],
        out_specs=out_specs,
        out_shape=out_shape,
    )(msum_p, cnt_p, fo, mask, b, wt, nmask)


def kernel(x, edge_attr, W1_ii, b1_ii, W2_ii, b2_ii, W1_uiu, b1_uiu,
           edge_index, edge_mask_ii, edge_mask_uiu, node_mask_item):
    src_d = edge_index[0].reshape(NW, NSC_D, SCK_D, CH)
    dst_d = edge_index[1].reshape(NW, NSC_D, SCK_D, CH)
    ea_d = edge_attr.reshape(NW, NSC_D, SCK_D, CH)
    src_m = edge_index[0].reshape(NS, 1, NSC_M, SCK_M, CH)
    coff = (jnp.arange(NC, dtype=jnp.int32) * N).reshape(1, NC, 1, 1, 1)
    src_m = (src_m + coff).reshape(NW, NSC_M, SCK_M, CH)
    dst_m = edge_index[1].reshape(NS, NSC_M, SCK_M, CH)
    ea_m = edge_attr.reshape(NS, NSC_M, SCK_M, CH)
    mii_m = edge_mask_ii.astype(jnp.float32).reshape(NS, NSC_M, SCK_M, CH)
    muiu_m = edge_mask_uiu.astype(jnp.float32).reshape(NS, NSC_M, SCK_M, CH)
    muiu_d = edge_mask_uiu.astype(jnp.float32).reshape(NW, NSC_D, SCK_D, CH)
    nmask = node_mask_item.astype(jnp.float32)[:, None]
    allmask = jnp.ones((N, 1), jnp.float32)

    b1i = b1_ii.reshape(1, D)
    b2i = b2_ii.reshape(1, D)
    b1u = b1_uiu.reshape(1, D)
    w1t = W1_ii.T
    w2t = W2_ii.T
    wut = W1_uiu.T

    un0, fo1 = _prep(x, w1t, nmask)
    w1 = _edge_dot(un0, src_d, dst_d, ea_d)
    ms1, ct_ii = _edge_msg(fo1.reshape(2 * N, HD), src_m, dst_m,
                           w1.reshape(NS, NSC_M, SCK_M, CH), mii_m,
                           with_cnt=True)
    fo2, un1 = _post(ms1, ct_ii, fo1, nmask, b1i, w2t, nmask,
                     need_un=True, need_fo=True)
    w2 = _edge_dot(un1, src_d, dst_d, ea_d)
    ms2 = _edge_msg(fo2.reshape(2 * N, HD), src_m, dst_m,
                    w2.reshape(NS, NSC_M, SCK_M, CH), mii_m, with_cnt=False)
    (fo3,) = _post(ms2, ct_ii, fo2, nmask, b2i, wut, allmask,
                   need_un=False, need_fo=True)
    ms3, ct_uiu = _edge_msg(fo3.reshape(2 * N, HD), src_m, dst_m, ea_m,
                            muiu_m, with_cnt=True)
    (fo4,) = _post(ms3, ct_uiu, fo3, allmask, b1u, wut, allmask,
                   need_un=False, need_fo=True)
    ms4 = _edge_msg(fo4.reshape(2 * N, HD), src_m, dst_m, ea_m, muiu_m,
                    with_cnt=False)
    (un4,) = _post(ms4, ct_uiu, fo4, allmask, b1u, wut, allmask,
                   need_un=True, need_fo=False)
    y = _edge_dot(un4, src_d, dst_d, muiu_d)
    return y.reshape(E)
